# Initial kernel scaffold; baseline (speedup 1.0000x reference)
import jax
import jax.numpy as jnp
from jax.experimental import pallas as pl


def kernel(query, query_pos, value, reference_points, spatial_shapes, level_start_index, score_tgt, foreground_pre_layer, W_val, b_val, W_off, b_off, W_attw, b_attw, W_out, b_out, g1, be1, W_l1, b_l1, W_l2, b_l2, g2, be2):
    raise NotImplementedError("write your pallas kernel here")


# speedup vs baseline: 37.6141x; 37.6141x over previous
import functools

import jax
import jax.numpy as jnp
import numpy as np
from jax import lax
from jax.experimental import pallas as pl
from jax.experimental.pallas import tpu as pltpu
from jax.experimental.pallas import tpu_sc as plsc

_B = 1
_C = 256
_NH = 8
_NL = 4
_NP = 4
_DH = _C // _NH
_DFFN = 1024
_SPATIAL = [(100, 100), (50, 50), (25, 25), (13, 13)]
_N = sum(h * w for h, w in _SPATIAL)

_RB = 512
_NBLK = 26
_NPAD = _RB * _NBLK
_NW = 32
_QPW = _NPAD // _NW
_CQ = 2
_NCHUNK = _QPW // _CQ


def _lane_tables():
    lvl = np.array([(j // 4) % 4 for j in range(128)])
    wf = np.array([_SPATIAL[l][1] for l in lvl], np.float32)
    hf = np.array([_SPATIAL[l][0] for l in lvl], np.float32)
    wi = wf.astype(np.int32)
    starts = np.cumsum([0] + [h * w for h, w in _SPATIAL])[:_NL]
    st = np.array([starts[l] for l in lvl], np.int32)
    hd = np.array([j // 16 for j in range(128)], np.int32)
    g = np.zeros((128, 128), np.float32)
    for a in range(128):
        g[a, (a // 16) * 16:(a // 16) * 16 + 16] = 1.0
    px = np.zeros((8, 128), np.float32)
    py = np.zeros((8, 128), np.float32)
    for j in range(128):
        px[2 * lvl[j], j] = 1.0
        py[2 * lvl[j] + 1, j] = 1.0
    return (wf.reshape(1, 128), hf.reshape(1, 128), wi.reshape(1, 128),
            st.reshape(1, 128), hd.reshape(1, 128), g, px, py)


_WF, _HF, _WI, _ST, _HD, _G, _PX, _PY = _lane_tables()


def _prep_body(q_ref, qp_ref, v_ref, rp_ref,
               wval_ref, bval_ref, wx_ref, bx_ref, wy_ref, by_ref,
               wa_ref, ba_ref, g_ref, px_ref, py_ref,
               wf_ref, hf_ref, wi_ref, st_ref, hd_ref,
               vp_ref, i0_ref, i1_ref, i2_ref, i3_ref,
               w0_ref, w1_ref, w2_ref, w3_ref):
    f32 = jnp.float32
    i = pl.program_id(0)
    q_in = q_ref[...] + qp_ref[...]
    vp_ref[...] = (jnp.dot(v_ref[...], wval_ref[...], preferred_element_type=f32)
                   + bval_ref[...])
    offx = jnp.dot(q_in, wx_ref[...], preferred_element_type=f32) + bx_ref[...]
    offy = jnp.dot(q_in, wy_ref[...], preferred_element_type=f32) + by_ref[...]
    logit = jnp.dot(q_in, wa_ref[...], preferred_element_type=f32) + ba_ref[...]
    e = jnp.exp(logit)
    s = jnp.dot(e, g_ref[...], preferred_element_type=f32)
    aw = e / s
    rx = jnp.dot(rp_ref[...], px_ref[...], preferred_element_type=f32)
    ry = jnp.dot(rp_ref[...], py_ref[...], preferred_element_type=f32)
    wf = wf_ref[...]
    hf = hf_ref[...]
    x = (rx + offx / wf) * wf - 0.5
    y = (ry + offy / hf) * hf - 0.5
    x0 = jnp.floor(x)
    y0 = jnp.floor(y)
    rows_valid = (lax.broadcasted_iota(jnp.int32, x.shape, 0) + i * _RB) < _N
    wi = wi_ref[...]
    st = st_ref[...]
    hd = hd_ref[...]
    outs = ((i0_ref, w0_ref), (i1_ref, w1_ref), (i2_ref, w2_ref), (i3_ref, w3_ref))
    c = 0
    for dy in (0, 1):
        for dx in (0, 1):
            xi = x0 + dx
            yi = y0 + dy
            wgt = (1.0 - jnp.abs(x - xi)) * (1.0 - jnp.abs(y - yi))
            valid = ((xi >= 0) & (xi <= wf - 1) & (yi >= 0) & (yi <= hf - 1)
                     & rows_valid)
            cxi = jnp.clip(xi, 0, wf - 1).astype(jnp.int32)
            cyi = jnp.clip(yi, 0, hf - 1).astype(jnp.int32)
            pix = st + cyi * wi + cxi
            ir, wr = outs[c]
            ir[...] = jnp.where(rows_valid, pix * _NH + hd, 0)
            wr[...] = jnp.where(valid, wgt * aw, 0.0)
            c += 1


def _sc_body(v2, i0, i1, i2, i3, w0, w1, w2, w3, out,
             idxs, wgts, rows, outv, gsem):
    wid = lax.axis_index("s") * 2 + lax.axis_index("c")
    base0 = wid * _QPW
    iota16 = lax.iota(jnp.int32, 16)
    iota16p = iota16 + 16
    idx_hbm = (i0, i1, i2, i3)
    wgt_hbm = (w0, w1, w2, w3)

    def chunk(g, carry):
        qb = base0 + g * _CQ
        for c in range(4):
            pltpu.sync_copy(idx_hbm[c].at[pl.ds(qb, _CQ)], idxs.at[c])
            pltpu.sync_copy(wgt_hbm[c].at[pl.ds(qb, _CQ)], wgts.at[c])
        handles = []
        for c in range(4):
            for q in range(_CQ):
                handles.append(pltpu.async_copy(v2.at[idxs.at[c, q]],
                                                rows.at[c, q], gsem))
        for h in handles:
            h.wait()
        for q in range(_CQ):
            def hbody(hh, carry2):
                acc0 = jnp.zeros((16,), jnp.float32)
                acc1 = jnp.zeros((16,), jnp.float32)
                for c in range(4):
                    wref = wgts.at[c, q]
                    rref = rows.at[c, q]
                    for k in range(16):
                        jv = jnp.full((16,), hh * 16 + k, jnp.int32)
                        r0 = plsc.load_gather(rref, [jv, iota16])
                        r1 = plsc.load_gather(rref, [jv, iota16p])
                        wb = plsc.load_gather(wref, [jv])
                        acc0 = acc0 + wb * r0
                        acc1 = acc1 + wb * r1
                outv[pl.ds(q * _C + hh * 32, 16)] = acc0
                outv[pl.ds(q * _C + hh * 32 + 16, 16)] = acc1
                return carry2
            lax.fori_loop(0, _NH, hbody, 0)
        pltpu.sync_copy(outv, out.at[pl.ds(qb * _C, _CQ * _C)])
        return carry

    lax.fori_loop(0, _NCHUNK, chunk, 0)


def _sc_msda(v2, i0, i1, i2, i3, w0, w1, w2, w3):
    mesh = plsc.VectorSubcoreMesh(core_axis_name="c", subcore_axis_name="s")
    return pl.kernel(
        _sc_body,
        out_type=jax.ShapeDtypeStruct((_NPAD * _C,), jnp.float32),
        mesh=mesh,
        compiler_params=pltpu.CompilerParams(needs_layout_passes=False,
                                             use_tc_tiling_on_sc=False),
        scratch_types=[
            pltpu.VMEM((4, _CQ, 128), jnp.int32),
            pltpu.VMEM((4, _CQ, 128), jnp.float32),
            pltpu.VMEM((4, _CQ, 128, _DH), jnp.float32),
            pltpu.VMEM((_CQ * _C,), jnp.float32),
            pltpu.SemaphoreType.DMA,
        ],
    )(v2, i0, i1, i2, i3, w0, w1, w2, w3)


def _tail_body(ms_ref, q_ref, wo_ref, bo_ref, g1_ref, be1_ref,
               wl1_ref, bl1_ref, wl2_ref, bl2_ref, g2_ref, be2_ref, o_ref):
    f32 = jnp.float32
    src2 = jnp.dot(ms_ref[...], wo_ref[...], preferred_element_type=f32) + bo_ref[...]
    x = q_ref[...] + src2
    m = jnp.mean(x, axis=-1, keepdims=True)
    d = x - m
    v = jnp.mean(d * d, axis=-1, keepdims=True)
    q1 = d / jnp.sqrt(v + 1e-5) * g1_ref[...] + be1_ref[...]
    h1 = jnp.maximum(jnp.dot(q1, wl1_ref[...], preferred_element_type=f32)
                     + bl1_ref[...], 0.0)
    y = q1 + jnp.dot(h1, wl2_ref[...], preferred_element_type=f32) + bl2_ref[...]
    m2 = jnp.mean(y, axis=-1, keepdims=True)
    d2 = y - m2
    v2 = jnp.mean(d2 * d2, axis=-1, keepdims=True)
    o_ref[...] = d2 / jnp.sqrt(v2 + 1e-5) * g2_ref[...] + be2_ref[...]


def _full(shape):
    return pl.BlockSpec(shape, lambda i: tuple(0 for _ in shape))


def _prep_call(q, qp, v, rp8, W_val, b_val, Wx, bx, Wy, by, W_attw, b_attw):
    row = lambda i: (i, 0)
    outs = [jax.ShapeDtypeStruct((_N, _C), jnp.float32)]
    outs += [jax.ShapeDtypeStruct((_NPAD, 128), jnp.int32)] * 4
    outs += [jax.ShapeDtypeStruct((_NPAD, 128), jnp.float32)] * 4
    out_specs = [pl.BlockSpec((_RB, _C), row)] + [pl.BlockSpec((_RB, 128), row)] * 8
    in_specs = [pl.BlockSpec((_RB, _C), row)] * 3 + [pl.BlockSpec((_RB, 8), row)]
    in_specs += [_full(a.shape) for a in
                 (W_val, b_val, Wx, bx, Wy, by, W_attw, b_attw,
                  _G, _PX, _PY, _WF, _HF, _WI, _ST, _HD)]
    return pl.pallas_call(
        _prep_body, grid=(_NBLK,), in_specs=in_specs,
        out_specs=out_specs, out_shape=outs,
    )(q, qp, v, rp8, W_val, b_val, Wx, bx, Wy, by, W_attw, b_attw,
      _G, _PX, _PY, _WF, _HF, _WI, _ST, _HD)


def _tail_call(ms, q, W_out, b_out, g1, be1, W_l1, b_l1, W_l2, b_l2, g2, be2):
    row = lambda i: (i, 0)
    full_args = (W_out, b_out, g1, be1, W_l1, b_l1, W_l2, b_l2, g2, be2)
    in_specs = [pl.BlockSpec((_RB, _C), row)] * 2 + [_full(a.shape) for a in full_args]
    return pl.pallas_call(
        _tail_body, grid=(_NBLK,), in_specs=in_specs,
        out_specs=pl.BlockSpec((_RB, _C), row),
        out_shape=jax.ShapeDtypeStruct((_N, _C), jnp.float32),
    )(ms, q, *full_args)


def kernel(query, query_pos, value, reference_points, spatial_shapes,
           level_start_index, score_tgt, foreground_pre_layer,
           W_val, b_val, W_off, b_off, W_attw, b_attw, W_out, b_out,
           g1, be1, W_l1, b_l1, W_l2, b_l2, g2, be2):
    q = query.reshape(_N, _C)
    qp = query_pos.reshape(_N, _C)
    v = value.reshape(_N, _C)
    rp8 = reference_points.reshape(_N, _NL * 2)
    Wx = W_off[:, 0::2]
    bx = b_off[0::2].reshape(1, 128)
    Wy = W_off[:, 1::2]
    by = b_off[1::2].reshape(1, 128)
    vp, i0, i1, i2, i3, w0, w1, w2, w3 = _prep_call(
        q, qp, v, rp8, W_val, b_val.reshape(1, _C), Wx, bx, Wy, by,
        W_attw, b_attw.reshape(1, 128))
    v2 = vp.reshape(_N * _NH, _DH)
    ms = _sc_msda(v2, i0, i1, i2, i3, w0, w1, w2, w3).reshape(_NPAD, _C)
    out = _tail_call(ms, q, W_out, b_out.reshape(1, _C),
                     g1.reshape(1, _C), be1.reshape(1, _C),
                     W_l1, b_l1.reshape(1, _DFFN), W_l2, b_l2.reshape(1, _C),
                     g2.reshape(1, _C), be2.reshape(1, _C))
    return out.reshape(_B, _N, _C)


# speedup vs baseline: 42.5522x; 1.1313x over previous
import functools

import jax
import jax.numpy as jnp
import numpy as np
from jax import lax
from jax.experimental import pallas as pl
from jax.experimental.pallas import tpu as pltpu
from jax.experimental.pallas import tpu_sc as plsc

_B = 1
_C = 256
_NH = 8
_NL = 4
_NP = 4
_DH = _C // _NH
_DFFN = 1024
_SPATIAL = [(100, 100), (50, 50), (25, 25), (13, 13)]
_N = sum(h * w for h, w in _SPATIAL)

_RB = 512
_NBLK = 26
_NPAD = _RB * _NBLK
_NW = 32
_QPW = _NPAD // _NW
_CQ = 2
_NCHUNK = _QPW // _CQ


def _lane_tables():
    lvl = np.array([(j // 4) % 4 for j in range(128)])
    wf = np.array([_SPATIAL[l][1] for l in lvl], np.float32)
    hf = np.array([_SPATIAL[l][0] for l in lvl], np.float32)
    wi = wf.astype(np.int32)
    starts = np.cumsum([0] + [h * w for h, w in _SPATIAL])[:_NL]
    st = np.array([starts[l] for l in lvl], np.int32)
    hd = np.array([j // 16 for j in range(128)], np.int32)
    g = np.zeros((128, 128), np.float32)
    for a in range(128):
        g[a, (a // 16) * 16:(a // 16) * 16 + 16] = 1.0
    px = np.zeros((8, 128), np.float32)
    py = np.zeros((8, 128), np.float32)
    for j in range(128):
        px[2 * lvl[j], j] = 1.0
        py[2 * lvl[j] + 1, j] = 1.0
    return (wf.reshape(1, 128), hf.reshape(1, 128), wi.reshape(1, 128),
            st.reshape(1, 128), hd.reshape(1, 128), g, px, py)


_WF, _HF, _WI, _ST, _HD, _G, _PX, _PY = _lane_tables()


def _prep_body(q_ref, qp_ref, v_ref, rp_ref,
               wval_ref, bval_ref, wx_ref, bx_ref, wy_ref, by_ref,
               wa_ref, ba_ref, g_ref, px_ref, py_ref,
               wf_ref, hf_ref, wi_ref, st_ref, hd_ref,
               vp_ref, i0_ref, i1_ref, i2_ref, i3_ref,
               w0_ref, w1_ref, w2_ref, w3_ref):
    f32 = jnp.float32
    i = pl.program_id(0)
    q_in = q_ref[...] + qp_ref[...]
    vp_ref[...] = (jnp.dot(v_ref[...], wval_ref[...], preferred_element_type=f32)
                   + bval_ref[...])
    offx = jnp.dot(q_in, wx_ref[...], preferred_element_type=f32) + bx_ref[...]
    offy = jnp.dot(q_in, wy_ref[...], preferred_element_type=f32) + by_ref[...]
    logit = jnp.dot(q_in, wa_ref[...], preferred_element_type=f32) + ba_ref[...]
    e = jnp.exp(logit)
    s = jnp.dot(e, g_ref[...], preferred_element_type=f32)
    aw = e / s
    rx = jnp.dot(rp_ref[...], px_ref[...], preferred_element_type=f32)
    ry = jnp.dot(rp_ref[...], py_ref[...], preferred_element_type=f32)
    wf = wf_ref[...]
    hf = hf_ref[...]
    x = (rx + offx / wf) * wf - 0.5
    y = (ry + offy / hf) * hf - 0.5
    x0 = jnp.floor(x)
    y0 = jnp.floor(y)
    rows_valid = (lax.broadcasted_iota(jnp.int32, x.shape, 0) + i * _RB) < _N
    wi = wi_ref[...]
    st = st_ref[...]
    hd = hd_ref[...]
    outs = ((i0_ref, w0_ref), (i1_ref, w1_ref), (i2_ref, w2_ref), (i3_ref, w3_ref))
    c = 0
    for dy in (0, 1):
        for dx in (0, 1):
            xi = x0 + dx
            yi = y0 + dy
            wgt = (1.0 - jnp.abs(x - xi)) * (1.0 - jnp.abs(y - yi))
            valid = ((xi >= 0) & (xi <= wf - 1) & (yi >= 0) & (yi <= hf - 1)
                     & rows_valid)
            cxi = jnp.clip(xi, 0, wf - 1).astype(jnp.int32)
            cyi = jnp.clip(yi, 0, hf - 1).astype(jnp.int32)
            pix = st + cyi * wi + cxi
            ir, wr = outs[c]
            ir[...] = jnp.where(rows_valid, pix * _NH + hd, 0)
            wr[...] = jnp.where(valid, wgt * aw, 0.0)
            c += 1


def _sc_body(v2, i0, i1, i2, i3, w0, w1, w2, w3, out,
             idxs, wgts, rows, outv, gsem):
    wid = lax.axis_index("s") * 2 + lax.axis_index("c")
    base0 = wid * _QPW
    iota16 = lax.iota(jnp.int32, 16)
    iota16p = iota16 + 16
    idx_hbm = (i0, i1, i2, i3)
    wgt_hbm = (w0, w1, w2, w3)

    def chunk(g, carry):
        qb = base0 + g * _CQ
        for c in range(4):
            pltpu.sync_copy(idx_hbm[c].at[pl.ds(qb, _CQ)], idxs.at[c])
            pltpu.sync_copy(wgt_hbm[c].at[pl.ds(qb, _CQ)], wgts.at[c])
        handles = []
        for c in range(4):
            for q in range(_CQ):
                handles.append(pltpu.async_copy(v2.at[idxs.at[c, q]],
                                                rows.at[c, q], gsem))
        for h in handles:
            h.wait()
        for q in range(_CQ):
            def hbody(hh, carry2):
                acc0 = jnp.zeros((16,), jnp.float32)
                acc1 = jnp.zeros((16,), jnp.float32)
                hb = hh * 16
                for c in range(4):
                    wv = wgts[c, q, pl.ds(hb, 16)]
                    for k in range(16):
                        r0 = rows[c, q, hb + k, pl.ds(0, 16)]
                        r1 = rows[c, q, hb + k, pl.ds(16, 16)]
                        wb = wv[k]
                        acc0 = acc0 + wb * r0
                        acc1 = acc1 + wb * r1
                outv[pl.ds(q * _C + hh * 32, 16)] = acc0
                outv[pl.ds(q * _C + hh * 32 + 16, 16)] = acc1
                return carry2
            lax.fori_loop(0, _NH, hbody, 0)
        pltpu.sync_copy(outv, out.at[pl.ds(qb * _C, _CQ * _C)])
        return carry

    lax.fori_loop(0, _NCHUNK, chunk, 0)


def _sc_msda(v2, i0, i1, i2, i3, w0, w1, w2, w3):
    mesh = plsc.VectorSubcoreMesh(core_axis_name="c", subcore_axis_name="s")
    return pl.kernel(
        _sc_body,
        out_type=jax.ShapeDtypeStruct((_NPAD * _C,), jnp.float32),
        mesh=mesh,
        compiler_params=pltpu.CompilerParams(needs_layout_passes=False,
                                             use_tc_tiling_on_sc=False),
        scratch_types=[
            pltpu.VMEM((4, _CQ, 128), jnp.int32),
            pltpu.VMEM((4, _CQ, 128), jnp.float32),
            pltpu.VMEM((4, _CQ, 128, _DH), jnp.float32),
            pltpu.VMEM((_CQ * _C,), jnp.float32),
            pltpu.SemaphoreType.DMA,
        ],
    )(v2, i0, i1, i2, i3, w0, w1, w2, w3)


def _tail_body(ms_ref, q_ref, wo_ref, bo_ref, g1_ref, be1_ref,
               wl1_ref, bl1_ref, wl2_ref, bl2_ref, g2_ref, be2_ref, o_ref):
    f32 = jnp.float32
    src2 = jnp.dot(ms_ref[...], wo_ref[...], preferred_element_type=f32) + bo_ref[...]
    x = q_ref[...] + src2
    m = jnp.mean(x, axis=-1, keepdims=True)
    d = x - m
    v = jnp.mean(d * d, axis=-1, keepdims=True)
    q1 = d / jnp.sqrt(v + 1e-5) * g1_ref[...] + be1_ref[...]
    h1 = jnp.maximum(jnp.dot(q1, wl1_ref[...], preferred_element_type=f32)
                     + bl1_ref[...], 0.0)
    y = q1 + jnp.dot(h1, wl2_ref[...], preferred_element_type=f32) + bl2_ref[...]
    m2 = jnp.mean(y, axis=-1, keepdims=True)
    d2 = y - m2
    v2 = jnp.mean(d2 * d2, axis=-1, keepdims=True)
    o_ref[...] = d2 / jnp.sqrt(v2 + 1e-5) * g2_ref[...] + be2_ref[...]


def _full(shape):
    return pl.BlockSpec(shape, lambda i: tuple(0 for _ in shape))


def _prep_call(q, qp, v, rp8, W_val, b_val, Wx, bx, Wy, by, W_attw, b_attw):
    row = lambda i: (i, 0)
    outs = [jax.ShapeDtypeStruct((_N, _C), jnp.float32)]
    outs += [jax.ShapeDtypeStruct((_NPAD, 128), jnp.int32)] * 4
    outs += [jax.ShapeDtypeStruct((_NPAD, 128), jnp.float32)] * 4
    out_specs = [pl.BlockSpec((_RB, _C), row)] + [pl.BlockSpec((_RB, 128), row)] * 8
    in_specs = [pl.BlockSpec((_RB, _C), row)] * 3 + [pl.BlockSpec((_RB, 8), row)]
    in_specs += [_full(a.shape) for a in
                 (W_val, b_val, Wx, bx, Wy, by, W_attw, b_attw,
                  _G, _PX, _PY, _WF, _HF, _WI, _ST, _HD)]
    return pl.pallas_call(
        _prep_body, grid=(_NBLK,), in_specs=in_specs,
        out_specs=out_specs, out_shape=outs,
    )(q, qp, v, rp8, W_val, b_val, Wx, bx, Wy, by, W_attw, b_attw,
      _G, _PX, _PY, _WF, _HF, _WI, _ST, _HD)


def _tail_call(ms, q, W_out, b_out, g1, be1, W_l1, b_l1, W_l2, b_l2, g2, be2):
    row = lambda i: (i, 0)
    full_args = (W_out, b_out, g1, be1, W_l1, b_l1, W_l2, b_l2, g2, be2)
    in_specs = [pl.BlockSpec((_RB, _C), row)] * 2 + [_full(a.shape) for a in full_args]
    return pl.pallas_call(
        _tail_body, grid=(_NBLK,), in_specs=in_specs,
        out_specs=pl.BlockSpec((_RB, _C), row),
        out_shape=jax.ShapeDtypeStruct((_N, _C), jnp.float32),
    )(ms, q, *full_args)


def kernel(query, query_pos, value, reference_points, spatial_shapes,
           level_start_index, score_tgt, foreground_pre_layer,
           W_val, b_val, W_off, b_off, W_attw, b_attw, W_out, b_out,
           g1, be1, W_l1, b_l1, W_l2, b_l2, g2, be2):
    q = query.reshape(_N, _C)
    qp = query_pos.reshape(_N, _C)
    v = value.reshape(_N, _C)
    rp8 = reference_points.reshape(_N, _NL * 2)
    Wx = W_off[:, 0::2]
    bx = b_off[0::2].reshape(1, 128)
    Wy = W_off[:, 1::2]
    by = b_off[1::2].reshape(1, 128)
    vp, i0, i1, i2, i3, w0, w1, w2, w3 = _prep_call(
        q, qp, v, rp8, W_val, b_val.reshape(1, _C), Wx, bx, Wy, by,
        W_attw, b_attw.reshape(1, 128))
    v2 = vp.reshape(_N * _NH, _DH)
    ms = _sc_msda(v2, i0, i1, i2, i3, w0, w1, w2, w3).reshape(_NPAD, _C)
    out = _tail_call(ms, q, W_out, b_out.reshape(1, _C),
                     g1.reshape(1, _C), be1.reshape(1, _C),
                     W_l1, b_l1.reshape(1, _DFFN), W_l2, b_l2.reshape(1, _C),
                     g2.reshape(1, _C), be2.reshape(1, _C))
    return out.reshape(_B, _N, _C)


# speedup vs baseline: 90.0892x; 2.1171x over previous
import functools

import jax
import jax.numpy as jnp
import numpy as np
from jax import lax
from jax.experimental import pallas as pl
from jax.experimental.pallas import tpu as pltpu
from jax.experimental.pallas import tpu_sc as plsc

_B = 1
_C = 256
_NH = 8
_NL = 4
_NP = 4
_DH = _C // _NH
_DFFN = 1024
_SPATIAL = [(100, 100), (50, 50), (25, 25), (13, 13)]
_N = sum(h * w for h, w in _SPATIAL)

_RB = 512
_NBLK = 26
_NPAD = _RB * _NBLK
_NW = 32
_QPW = _NPAD // _NW
_CQ = 2
_NCHUNK = _QPW // _CQ


def _lane_tables():
    lvl = np.array([(j // 4) % 4 for j in range(128)])
    wf = np.array([_SPATIAL[l][1] for l in lvl], np.float32)
    hf = np.array([_SPATIAL[l][0] for l in lvl], np.float32)
    wi = wf.astype(np.int32)
    starts = np.cumsum([0] + [h * w for h, w in _SPATIAL])[:_NL]
    st = np.array([starts[l] for l in lvl], np.int32)
    hd = np.array([j // 16 for j in range(128)], np.int32)
    g = np.zeros((128, 128), np.float32)
    for a in range(128):
        g[a, (a // 16) * 16:(a // 16) * 16 + 16] = 1.0
    px = np.zeros((8, 128), np.float32)
    py = np.zeros((8, 128), np.float32)
    for j in range(128):
        px[2 * lvl[j], j] = 1.0
        py[2 * lvl[j] + 1, j] = 1.0
    return (wf.reshape(1, 128), hf.reshape(1, 128), wi.reshape(1, 128),
            st.reshape(1, 128), hd.reshape(1, 128), g, px, py)


_WF, _HF, _WI, _ST, _HD, _G, _PX, _PY = _lane_tables()


def _prep_body(q_ref, qp_ref, v_ref, rp_ref,
               wval_ref, bval_ref, wx_ref, bx_ref, wy_ref, by_ref,
               wa_ref, ba_ref, g_ref, px_ref, py_ref,
               wf_ref, hf_ref, wi_ref, st_ref, hd_ref,
               vp_ref, i0_ref, i1_ref, i2_ref, i3_ref,
               w0_ref, w1_ref, w2_ref, w3_ref):
    f32 = jnp.float32
    i = pl.program_id(0)
    q_in = q_ref[...] + qp_ref[...]
    vp_ref[...] = (jnp.dot(v_ref[...], wval_ref[...], preferred_element_type=f32)
                   + bval_ref[...])
    offx = jnp.dot(q_in, wx_ref[...], preferred_element_type=f32) + bx_ref[...]
    offy = jnp.dot(q_in, wy_ref[...], preferred_element_type=f32) + by_ref[...]
    logit = jnp.dot(q_in, wa_ref[...], preferred_element_type=f32) + ba_ref[...]
    e = jnp.exp(logit)
    s = jnp.dot(e, g_ref[...], preferred_element_type=f32)
    aw = e / s
    rx = jnp.dot(rp_ref[...], px_ref[...], preferred_element_type=f32)
    ry = jnp.dot(rp_ref[...], py_ref[...], preferred_element_type=f32)
    wf = wf_ref[...]
    hf = hf_ref[...]
    x = (rx + offx / wf) * wf - 0.5
    y = (ry + offy / hf) * hf - 0.5
    x0 = jnp.floor(x)
    y0 = jnp.floor(y)
    rows_valid = (lax.broadcasted_iota(jnp.int32, x.shape, 0) + i * _RB) < _N
    wi = wi_ref[...]
    st = st_ref[...]
    hd = hd_ref[...]
    outs = ((i0_ref, w0_ref), (i1_ref, w1_ref), (i2_ref, w2_ref), (i3_ref, w3_ref))
    c = 0
    for dy in (0, 1):
        for dx in (0, 1):
            xi = x0 + dx
            yi = y0 + dy
            wgt = (1.0 - jnp.abs(x - xi)) * (1.0 - jnp.abs(y - yi))
            valid = ((xi >= 0) & (xi <= wf - 1) & (yi >= 0) & (yi <= hf - 1)
                     & rows_valid)
            cxi = jnp.clip(xi, 0, wf - 1).astype(jnp.int32)
            cyi = jnp.clip(yi, 0, hf - 1).astype(jnp.int32)
            pix = st + cyi * wi + cxi
            ir, wr = outs[c]
            ir[...] = jnp.where(rows_valid, pix * _NH + hd, 0)
            wr[...] = jnp.where(valid, wgt * aw, 0.0)
            c += 1


def _sc_body(v2, i0, i1, i2, i3, w0, w1, w2, w3, out,
             idxs, wgts, rows, outv, ss0, ss1, gs0, gs1):
    wid = lax.axis_index("s") * 2 + lax.axis_index("c")
    base0 = wid * _QPW
    idx_hbm = (i0, i1, i2, i3)
    wgt_hbm = (w0, w1, w2, w3)
    ssems = (ss0, ss1)
    gsems = (gs0, gs1)

    def fire_stage(g, s):
        qb = base0 + g * _CQ
        for c in range(4):
            pltpu.async_copy(idx_hbm[c].at[pl.ds(qb, _CQ)], idxs.at[s, c],
                             ssems[s])
            pltpu.async_copy(wgt_hbm[c].at[pl.ds(qb, _CQ)], wgts.at[s, c],
                             ssems[s])

    def wait_stage(s):
        for c in range(4):
            pltpu.make_async_copy(idx_hbm[c].at[pl.ds(0, _CQ)],
                                  idxs.at[s, c], ssems[s]).wait()
            pltpu.make_async_copy(wgt_hbm[c].at[pl.ds(0, _CQ)],
                                  wgts.at[s, c], ssems[s]).wait()

    def fire_gathers(s):
        for c in range(4):
            for q in range(_CQ):
                pltpu.async_copy(v2.at[idxs.at[s, c, q]], rows.at[s, c, q],
                                 gsems[s])

    def wait_gathers(s):
        for c in range(4):
            for q in range(_CQ):
                pltpu.make_async_copy(v2.at[idxs.at[s, c, q]],
                                      rows.at[s, c, q], gsems[s]).wait()

    def compute(g, s):
        qb = base0 + g * _CQ
        for q in range(_CQ):
            def hbody(hh, carry2):
                acc0 = jnp.zeros((16,), jnp.float32)
                acc1 = jnp.zeros((16,), jnp.float32)
                hb = hh * 16
                for c in range(4):
                    wv = wgts[s, c, q, pl.ds(hb, 16)]
                    for k in range(16):
                        r0 = rows[s, c, q, hb + k, pl.ds(0, 16)]
                        r1 = rows[s, c, q, hb + k, pl.ds(16, 16)]
                        wb = wv[k]
                        acc0 = acc0 + wb * r0
                        acc1 = acc1 + wb * r1
                outv[pl.ds(q * _C + hh * 32, 16)] = acc0
                outv[pl.ds(q * _C + hh * 32 + 16, 16)] = acc1
                return carry2
            lax.fori_loop(0, _NH, hbody, 0)
        pltpu.sync_copy(outv, out.at[pl.ds(qb * _C, _CQ * _C)])

    fire_stage(0, 0)
    fire_stage(1, 1)
    wait_stage(0)
    fire_gathers(0)

    def pair(i, carry):
        for s in (0, 1):
            g = i * 2 + s
            o = 1 - s
            wait_gathers(s)

            @pl.when(g + 1 < _NCHUNK)
            def _():
                wait_stage(o)
                fire_gathers(o)

            compute(g, s)

            @pl.when(g + 2 < _NCHUNK)
            def _():
                fire_stage(g + 2, s)
        return carry

    lax.fori_loop(0, _NCHUNK // 2, pair, 0)


def _sc_msda(v2, i0, i1, i2, i3, w0, w1, w2, w3):
    mesh = plsc.VectorSubcoreMesh(core_axis_name="c", subcore_axis_name="s")
    return pl.kernel(
        _sc_body,
        out_type=jax.ShapeDtypeStruct((_NPAD * _C,), jnp.float32),
        mesh=mesh,
        compiler_params=pltpu.CompilerParams(needs_layout_passes=False,
                                             use_tc_tiling_on_sc=False),
        scratch_types=[
            pltpu.VMEM((2, 4, _CQ, 128), jnp.int32),
            pltpu.VMEM((2, 4, _CQ, 128), jnp.float32),
            pltpu.VMEM((2, 4, _CQ, 128, _DH), jnp.float32),
            pltpu.VMEM((_CQ * _C,), jnp.float32),
            pltpu.SemaphoreType.DMA,
            pltpu.SemaphoreType.DMA,
            pltpu.SemaphoreType.DMA,
            pltpu.SemaphoreType.DMA,
        ],
    )(v2, i0, i1, i2, i3, w0, w1, w2, w3)


def _tail_body(ms_ref, q_ref, wo_ref, bo_ref, g1_ref, be1_ref,
               wl1_ref, bl1_ref, wl2_ref, bl2_ref, g2_ref, be2_ref, o_ref):
    f32 = jnp.float32
    src2 = jnp.dot(ms_ref[...], wo_ref[...], preferred_element_type=f32) + bo_ref[...]
    x = q_ref[...] + src2
    m = jnp.mean(x, axis=-1, keepdims=True)
    d = x - m
    v = jnp.mean(d * d, axis=-1, keepdims=True)
    q1 = d / jnp.sqrt(v + 1e-5) * g1_ref[...] + be1_ref[...]
    h1 = jnp.maximum(jnp.dot(q1, wl1_ref[...], preferred_element_type=f32)
                     + bl1_ref[...], 0.0)
    y = q1 + jnp.dot(h1, wl2_ref[...], preferred_element_type=f32) + bl2_ref[...]
    m2 = jnp.mean(y, axis=-1, keepdims=True)
    d2 = y - m2
    v2 = jnp.mean(d2 * d2, axis=-1, keepdims=True)
    o_ref[...] = d2 / jnp.sqrt(v2 + 1e-5) * g2_ref[...] + be2_ref[...]


def _full(shape):
    return pl.BlockSpec(shape, lambda i: tuple(0 for _ in shape))


def _prep_call(q, qp, v, rp8, W_val, b_val, Wx, bx, Wy, by, W_attw, b_attw):
    row = lambda i: (i, 0)
    outs = [jax.ShapeDtypeStruct((_N, _C), jnp.float32)]
    outs += [jax.ShapeDtypeStruct((_NPAD, 128), jnp.int32)] * 4
    outs += [jax.ShapeDtypeStruct((_NPAD, 128), jnp.float32)] * 4
    out_specs = [pl.BlockSpec((_RB, _C), row)] + [pl.BlockSpec((_RB, 128), row)] * 8
    in_specs = [pl.BlockSpec((_RB, _C), row)] * 3 + [pl.BlockSpec((_RB, 8), row)]
    in_specs += [_full(a.shape) for a in
                 (W_val, b_val, Wx, bx, Wy, by, W_attw, b_attw,
                  _G, _PX, _PY, _WF, _HF, _WI, _ST, _HD)]
    return pl.pallas_call(
        _prep_body, grid=(_NBLK,), in_specs=in_specs,
        out_specs=out_specs, out_shape=outs,
    )(q, qp, v, rp8, W_val, b_val, Wx, bx, Wy, by, W_attw, b_attw,
      _G, _PX, _PY, _WF, _HF, _WI, _ST, _HD)


def _tail_call(ms, q, W_out, b_out, g1, be1, W_l1, b_l1, W_l2, b_l2, g2, be2):
    row = lambda i: (i, 0)
    full_args = (W_out, b_out, g1, be1, W_l1, b_l1, W_l2, b_l2, g2, be2)
    in_specs = [pl.BlockSpec((_RB, _C), row)] * 2 + [_full(a.shape) for a in full_args]
    return pl.pallas_call(
        _tail_body, grid=(_NBLK,), in_specs=in_specs,
        out_specs=pl.BlockSpec((_RB, _C), row),
        out_shape=jax.ShapeDtypeStruct((_N, _C), jnp.float32),
    )(ms, q, *full_args)


def kernel(query, query_pos, value, reference_points, spatial_shapes,
           level_start_index, score_tgt, foreground_pre_layer,
           W_val, b_val, W_off, b_off, W_attw, b_attw, W_out, b_out,
           g1, be1, W_l1, b_l1, W_l2, b_l2, g2, be2):
    q = query.reshape(_N, _C)
    qp = query_pos.reshape(_N, _C)
    v = value.reshape(_N, _C)
    rp8 = reference_points.reshape(_N, _NL * 2)
    Wx = W_off[:, 0::2]
    bx = b_off[0::2].reshape(1, 128)
    Wy = W_off[:, 1::2]
    by = b_off[1::2].reshape(1, 128)
    vp, i0, i1, i2, i3, w0, w1, w2, w3 = _prep_call(
        q, qp, v, rp8, W_val, b_val.reshape(1, _C), Wx, bx, Wy, by,
        W_attw, b_attw.reshape(1, 128))
    v2 = vp.reshape(_N * _NH, _DH)
    ms = _sc_msda(v2, i0, i1, i2, i3, w0, w1, w2, w3).reshape(_NPAD, _C)
    out = _tail_call(ms, q, W_out, b_out.reshape(1, _C),
                     g1.reshape(1, _C), be1.reshape(1, _C),
                     W_l1, b_l1.reshape(1, _DFFN), W_l2, b_l2.reshape(1, _C),
                     g2.reshape(1, _C), be2.reshape(1, _C))
    return out.reshape(_B, _N, _C)


# speedup vs baseline: 104.5975x; 1.1610x over previous
import functools

import jax
import jax.numpy as jnp
import numpy as np
from jax import lax
from jax.experimental import pallas as pl
from jax.experimental.pallas import tpu as pltpu
from jax.experimental.pallas import tpu_sc as plsc

_B = 1
_C = 256
_NH = 8
_NL = 4
_NP = 4
_DH = _C // _NH
_DFFN = 1024
_SPATIAL = [(100, 100), (50, 50), (25, 25), (13, 13)]
_N = sum(h * w for h, w in _SPATIAL)

_RB = 512
_NBLK = 26
_NPAD = _RB * _NBLK
_NW = 32
_QPW = _NPAD // _NW
_CQ = 2
_NCHUNK = _QPW // _CQ


def _lane_tables():
    lvl = np.array([(j // 4) % 4 for j in range(128)])
    wf = np.array([_SPATIAL[l][1] for l in lvl], np.float32)
    hf = np.array([_SPATIAL[l][0] for l in lvl], np.float32)
    wi = wf.astype(np.int32)
    starts = np.cumsum([0] + [h * w for h, w in _SPATIAL])[:_NL]
    st = np.array([starts[l] for l in lvl], np.int32)
    hd = np.array([j // 16 for j in range(128)], np.int32)
    g = np.zeros((128, 128), np.float32)
    for a in range(128):
        g[a, (a // 16) * 16:(a // 16) * 16 + 16] = 1.0
    px = np.zeros((8, 128), np.float32)
    py = np.zeros((8, 128), np.float32)
    for j in range(128):
        px[2 * lvl[j], j] = 1.0
        py[2 * lvl[j] + 1, j] = 1.0
    return (wf.reshape(1, 128), hf.reshape(1, 128), wi.reshape(1, 128),
            st.reshape(1, 128), hd.reshape(1, 128), g, px, py)


_WF, _HF, _WI, _ST, _HD, _G, _PX, _PY = _lane_tables()

_WOPERM = np.array(
    [(j // 32) * 32 + (2 * (j % 32) if j % 32 < 16 else 2 * (j % 32 - 16) + 1)
     for j in range(_C)], np.int32)


def _prep_body(q_ref, qp_ref, v_ref, rp_ref,
               wval_ref, bval_ref, wx_ref, bx_ref, wy_ref, by_ref,
               wa_ref, ba_ref, g_ref, px_ref, py_ref,
               wf_ref, hf_ref, wi_ref, st_ref, hd_ref,
               vp_ref, i0_ref, i1_ref, i2_ref, i3_ref,
               w0_ref, w1_ref, w2_ref, w3_ref):
    f32 = jnp.float32
    i = pl.program_id(0)
    q_in = q_ref[...] + qp_ref[...]
    vp_ref[...] = (jnp.dot(v_ref[...], wval_ref[...], preferred_element_type=f32)
                   + bval_ref[...]).astype(jnp.bfloat16)
    offx = jnp.dot(q_in, wx_ref[...], preferred_element_type=f32) + bx_ref[...]
    offy = jnp.dot(q_in, wy_ref[...], preferred_element_type=f32) + by_ref[...]
    logit = jnp.dot(q_in, wa_ref[...], preferred_element_type=f32) + ba_ref[...]
    e = jnp.exp(logit)
    s = jnp.dot(e, g_ref[...], preferred_element_type=f32)
    aw = e / s
    rx = jnp.dot(rp_ref[...], px_ref[...], preferred_element_type=f32)
    ry = jnp.dot(rp_ref[...], py_ref[...], preferred_element_type=f32)
    wf = wf_ref[...]
    hf = hf_ref[...]
    x = (rx + offx / wf) * wf - 0.5
    y = (ry + offy / hf) * hf - 0.5
    x0 = jnp.floor(x)
    y0 = jnp.floor(y)
    rows_valid = (lax.broadcasted_iota(jnp.int32, x.shape, 0) + i * _RB) < _N
    wi = wi_ref[...]
    st = st_ref[...]
    hd = hd_ref[...]
    outs = ((i0_ref, w0_ref), (i1_ref, w1_ref), (i2_ref, w2_ref), (i3_ref, w3_ref))
    c = 0
    for dy in (0, 1):
        for dx in (0, 1):
            xi = x0 + dx
            yi = y0 + dy
            wgt = (1.0 - jnp.abs(x - xi)) * (1.0 - jnp.abs(y - yi))
            valid = ((xi >= 0) & (xi <= wf - 1) & (yi >= 0) & (yi <= hf - 1)
                     & rows_valid)
            cxi = jnp.clip(xi, 0, wf - 1).astype(jnp.int32)
            cyi = jnp.clip(yi, 0, hf - 1).astype(jnp.int32)
            pix = st + cyi * wi + cxi
            ir, wr = outs[c]
            ir[...] = jnp.where(rows_valid, pix * _NH + hd, 0)
            wr[...] = jnp.where(valid, wgt * aw, 0.0)
            c += 1


def _sc_body(v2, i0, i1, i2, i3, w0, w1, w2, w3, out,
             idxs, wgts, rows, outv, ss0, ss1, gs0, gs1):
    wid = lax.axis_index("s") * 2 + lax.axis_index("c")
    base0 = wid * _QPW
    idx_hbm = (i0, i1, i2, i3)
    wgt_hbm = (w0, w1, w2, w3)
    ssems = (ss0, ss1)
    gsems = (gs0, gs1)

    def fire_stage(g, s):
        qb = base0 + g * _CQ
        for c in range(4):
            pltpu.async_copy(idx_hbm[c].at[pl.ds(qb, _CQ)], idxs.at[s, c],
                             ssems[s])
            pltpu.async_copy(wgt_hbm[c].at[pl.ds(qb, _CQ)], wgts.at[s, c],
                             ssems[s])

    def wait_stage(s):
        for c in range(4):
            pltpu.make_async_copy(idx_hbm[c].at[pl.ds(0, _CQ)],
                                  idxs.at[s, c], ssems[s]).wait()
            pltpu.make_async_copy(wgt_hbm[c].at[pl.ds(0, _CQ)],
                                  wgts.at[s, c], ssems[s]).wait()

    def fire_gathers(s):
        for c in range(4):
            for q in range(_CQ):
                pltpu.async_copy(v2.at[idxs.at[s, c, q]], rows.at[s, c, q],
                                 gsems[s])

    def wait_gathers(s):
        for c in range(4):
            for q in range(_CQ):
                pltpu.make_async_copy(v2.at[idxs.at[s, c, q]],
                                      rows.at[s, c, q], gsems[s]).wait()

    def compute(g, s):
        qb = base0 + g * _CQ
        for q in range(_CQ):
            def hbody(hh, carry2):
                acc0 = jnp.zeros((16,), jnp.float32)
                acc1 = jnp.zeros((16,), jnp.float32)
                hb = hh * 16
                for c in range(4):
                    wv = wgts[s, c, q, pl.ds(hb, 16)]
                    for k in range(16):
                        r = rows[s, c, q, hb + k, pl.ds(0, 32)]
                        r0, r1 = plsc.unpack(r, format=plsc.PackFormat.INTERLEAVED)
                        wb = wv[k]
                        acc0 = acc0 + wb * r0
                        acc1 = acc1 + wb * r1
                outv[pl.ds(q * _C + hh * 32, 16)] = acc0
                outv[pl.ds(q * _C + hh * 32 + 16, 16)] = acc1
                return carry2
            lax.fori_loop(0, _NH, hbody, 0)
        pltpu.sync_copy(outv, out.at[pl.ds(qb * _C, _CQ * _C)])

    fire_stage(0, 0)
    fire_stage(1, 1)
    wait_stage(0)
    fire_gathers(0)

    def pair(i, carry):
        for s in (0, 1):
            g = i * 2 + s
            o = 1 - s
            wait_gathers(s)

            @pl.when(g + 1 < _NCHUNK)
            def _():
                wait_stage(o)
                fire_gathers(o)

            compute(g, s)

            @pl.when(g + 2 < _NCHUNK)
            def _():
                fire_stage(g + 2, s)
        return carry

    lax.fori_loop(0, _NCHUNK // 2, pair, 0)


def _sc_msda(v2, i0, i1, i2, i3, w0, w1, w2, w3):
    mesh = plsc.VectorSubcoreMesh(core_axis_name="c", subcore_axis_name="s")
    return pl.kernel(
        _sc_body,
        out_type=jax.ShapeDtypeStruct((_NPAD * _C,), jnp.float32),
        mesh=mesh,
        compiler_params=pltpu.CompilerParams(needs_layout_passes=False,
                                             use_tc_tiling_on_sc=False),
        scratch_types=[
            pltpu.VMEM((2, 4, _CQ, 128), jnp.int32),
            pltpu.VMEM((2, 4, _CQ, 128), jnp.float32),
            pltpu.VMEM((2, 4, _CQ, 128, _DH), jnp.bfloat16),
            pltpu.VMEM((_CQ * _C,), jnp.float32),
            pltpu.SemaphoreType.DMA,
            pltpu.SemaphoreType.DMA,
            pltpu.SemaphoreType.DMA,
            pltpu.SemaphoreType.DMA,
        ],
    )(v2, i0, i1, i2, i3, w0, w1, w2, w3)


def _tail_body(ms_ref, q_ref, wo_ref, bo_ref, g1_ref, be1_ref,
               wl1_ref, bl1_ref, wl2_ref, bl2_ref, g2_ref, be2_ref, o_ref):
    f32 = jnp.float32
    src2 = jnp.dot(ms_ref[...], wo_ref[...], preferred_element_type=f32) + bo_ref[...]
    x = q_ref[...] + src2
    m = jnp.mean(x, axis=-1, keepdims=True)
    d = x - m
    v = jnp.mean(d * d, axis=-1, keepdims=True)
    q1 = d / jnp.sqrt(v + 1e-5) * g1_ref[...] + be1_ref[...]
    h1 = jnp.maximum(jnp.dot(q1, wl1_ref[...], preferred_element_type=f32)
                     + bl1_ref[...], 0.0)
    y = q1 + jnp.dot(h1, wl2_ref[...], preferred_element_type=f32) + bl2_ref[...]
    m2 = jnp.mean(y, axis=-1, keepdims=True)
    d2 = y - m2
    v2 = jnp.mean(d2 * d2, axis=-1, keepdims=True)
    o_ref[...] = d2 / jnp.sqrt(v2 + 1e-5) * g2_ref[...] + be2_ref[...]


def _full(shape):
    return pl.BlockSpec(shape, lambda i: tuple(0 for _ in shape))


def _prep_call(q, qp, v, rp8, W_val, b_val, Wx, bx, Wy, by, W_attw, b_attw):
    row = lambda i: (i, 0)
    outs = [jax.ShapeDtypeStruct((_N, _C), jnp.bfloat16)]
    outs += [jax.ShapeDtypeStruct((_NPAD, 128), jnp.int32)] * 4
    outs += [jax.ShapeDtypeStruct((_NPAD, 128), jnp.float32)] * 4
    out_specs = [pl.BlockSpec((_RB, _C), row)] + [pl.BlockSpec((_RB, 128), row)] * 8
    in_specs = [pl.BlockSpec((_RB, _C), row)] * 3 + [pl.BlockSpec((_RB, 8), row)]
    in_specs += [_full(a.shape) for a in
                 (W_val, b_val, Wx, bx, Wy, by, W_attw, b_attw,
                  _G, _PX, _PY, _WF, _HF, _WI, _ST, _HD)]
    return pl.pallas_call(
        _prep_body, grid=(_NBLK,), in_specs=in_specs,
        out_specs=out_specs, out_shape=outs,
    )(q, qp, v, rp8, W_val, b_val, Wx, bx, Wy, by, W_attw, b_attw,
      _G, _PX, _PY, _WF, _HF, _WI, _ST, _HD)


def _tail_call(ms, q, W_out, b_out, g1, be1, W_l1, b_l1, W_l2, b_l2, g2, be2):
    row = lambda i: (i, 0)
    full_args = (W_out, b_out, g1, be1, W_l1, b_l1, W_l2, b_l2, g2, be2)
    in_specs = [pl.BlockSpec((_RB, _C), row)] * 2 + [_full(a.shape) for a in full_args]
    return pl.pallas_call(
        _tail_body, grid=(_NBLK,), in_specs=in_specs,
        out_specs=pl.BlockSpec((_RB, _C), row),
        out_shape=jax.ShapeDtypeStruct((_N, _C), jnp.float32),
    )(ms, q, *full_args)


def kernel(query, query_pos, value, reference_points, spatial_shapes,
           level_start_index, score_tgt, foreground_pre_layer,
           W_val, b_val, W_off, b_off, W_attw, b_attw, W_out, b_out,
           g1, be1, W_l1, b_l1, W_l2, b_l2, g2, be2):
    q = query.reshape(_N, _C)
    qp = query_pos.reshape(_N, _C)
    v = value.reshape(_N, _C)
    rp8 = reference_points.reshape(_N, _NL * 2)
    Wx = W_off[:, 0::2]
    bx = b_off[0::2].reshape(1, 128)
    Wy = W_off[:, 1::2]
    by = b_off[1::2].reshape(1, 128)
    vp, i0, i1, i2, i3, w0, w1, w2, w3 = _prep_call(
        q, qp, v, rp8, W_val, b_val.reshape(1, _C), Wx, bx, Wy, by,
        W_attw, b_attw.reshape(1, 128))
    v2 = vp.reshape(_N * _NH, _DH)
    ms = _sc_msda(v2, i0, i1, i2, i3, w0, w1, w2, w3).reshape(_NPAD, _C)
    out = _tail_call(ms, q, W_out[_WOPERM], b_out.reshape(1, _C),
                     g1.reshape(1, _C), be1.reshape(1, _C),
                     W_l1, b_l1.reshape(1, _DFFN), W_l2, b_l2.reshape(1, _C),
                     g2.reshape(1, _C), be2.reshape(1, _C))
    return out.reshape(_B, _N, _C)


# speedup vs baseline: 113.0514x; 1.0808x over previous
import functools

import jax
import jax.numpy as jnp
import numpy as np
from jax import lax
from jax.experimental import pallas as pl
from jax.experimental.pallas import tpu as pltpu
from jax.experimental.pallas import tpu_sc as plsc

_B = 1
_C = 256
_NH = 8
_NL = 4
_NP = 4
_DH = _C // _NH
_DFFN = 1024
_SPATIAL = [(100, 100), (50, 50), (25, 25), (13, 13)]
_N = sum(h * w for h, w in _SPATIAL)

_RB = 512
_NBLK = 26
_NPAD = _RB * _NBLK
_NW = 32
_QPW = _NPAD // _NW
_CQ = 4
_NCHUNK = _QPW // _CQ


def _lane_tables():
    lvl = np.array([(j // 4) % 4 for j in range(128)])
    wf = np.array([_SPATIAL[l][1] for l in lvl], np.float32)
    hf = np.array([_SPATIAL[l][0] for l in lvl], np.float32)
    wi = wf.astype(np.int32)
    starts = np.cumsum([0] + [h * w for h, w in _SPATIAL])[:_NL]
    st = np.array([starts[l] for l in lvl], np.int32)
    hd = np.array([j // 16 for j in range(128)], np.int32)
    g = np.zeros((128, 128), np.float32)
    for a in range(128):
        g[a, (a // 16) * 16:(a // 16) * 16 + 16] = 1.0
    px = np.zeros((8, 128), np.float32)
    py = np.zeros((8, 128), np.float32)
    for j in range(128):
        px[2 * lvl[j], j] = 1.0
        py[2 * lvl[j] + 1, j] = 1.0
    return (wf.reshape(1, 128), hf.reshape(1, 128), wi.reshape(1, 128),
            st.reshape(1, 128), hd.reshape(1, 128), g, px, py)


_WF, _HF, _WI, _ST, _HD, _G, _PX, _PY = _lane_tables()

_WOPERM = np.array(
    [(j // 32) * 32 + (2 * (j % 32) if j % 32 < 16 else 2 * (j % 32 - 16) + 1)
     for j in range(_C)], np.int32)


def _prep_body(q_ref, qp_ref, v_ref, rp_ref,
               wval_ref, bval_ref, wx_ref, bx_ref, wy_ref, by_ref,
               wa_ref, ba_ref, g_ref, px_ref, py_ref,
               wf_ref, hf_ref, wi_ref, st_ref, hd_ref,
               vp_ref, i0_ref, i1_ref, i2_ref, i3_ref,
               w0_ref, w1_ref, w2_ref, w3_ref):
    f32 = jnp.float32
    i = pl.program_id(0)
    q_in = q_ref[...] + qp_ref[...]
    vp_ref[...] = (jnp.dot(v_ref[...], wval_ref[...], preferred_element_type=f32)
                   + bval_ref[...]).astype(jnp.bfloat16)
    offx = jnp.dot(q_in, wx_ref[...], preferred_element_type=f32) + bx_ref[...]
    offy = jnp.dot(q_in, wy_ref[...], preferred_element_type=f32) + by_ref[...]
    logit = jnp.dot(q_in, wa_ref[...], preferred_element_type=f32) + ba_ref[...]
    e = jnp.exp(logit)
    s = jnp.dot(e, g_ref[...], preferred_element_type=f32)
    aw = e / s
    rx = jnp.dot(rp_ref[...], px_ref[...], preferred_element_type=f32)
    ry = jnp.dot(rp_ref[...], py_ref[...], preferred_element_type=f32)
    wf = wf_ref[...]
    hf = hf_ref[...]
    x = (rx + offx / wf) * wf - 0.5
    y = (ry + offy / hf) * hf - 0.5
    x0 = jnp.floor(x)
    y0 = jnp.floor(y)
    rows_valid = (lax.broadcasted_iota(jnp.int32, x.shape, 0) + i * _RB) < _N
    wi = wi_ref[...]
    st = st_ref[...]
    hd = hd_ref[...]
    outs = ((i0_ref, w0_ref), (i1_ref, w1_ref), (i2_ref, w2_ref), (i3_ref, w3_ref))
    c = 0
    for dy in (0, 1):
        for dx in (0, 1):
            xi = x0 + dx
            yi = y0 + dy
            wgt = (1.0 - jnp.abs(x - xi)) * (1.0 - jnp.abs(y - yi))
            valid = ((xi >= 0) & (xi <= wf - 1) & (yi >= 0) & (yi <= hf - 1)
                     & rows_valid)
            cxi = jnp.clip(xi, 0, wf - 1).astype(jnp.int32)
            cyi = jnp.clip(yi, 0, hf - 1).astype(jnp.int32)
            pix = st + cyi * wi + cxi
            ir, wr = outs[c]
            ir[...] = jnp.where(rows_valid, pix * _NH + hd, 0)
            wr[...] = jnp.where(valid, wgt * aw, 0.0)
            c += 1


def _sc_body(v2, i0, i1, i2, i3, w0, w1, w2, w3, out,
             idxs, wgts, rows, outv, ss0, ss1, gs0, gs1):
    wid = lax.axis_index("s") * 2 + lax.axis_index("c")
    base0 = wid * _QPW
    idx_hbm = (i0, i1, i2, i3)
    wgt_hbm = (w0, w1, w2, w3)
    ssems = (ss0, ss1)
    gsems = (gs0, gs1)

    def fire_stage(g, s):
        qb = base0 + g * _CQ
        for c in range(4):
            pltpu.async_copy(idx_hbm[c].at[pl.ds(qb, _CQ)], idxs.at[s, c],
                             ssems[s])
            pltpu.async_copy(wgt_hbm[c].at[pl.ds(qb, _CQ)], wgts.at[s, c],
                             ssems[s])

    def wait_stage(s):
        for c in range(4):
            pltpu.make_async_copy(idx_hbm[c].at[pl.ds(0, _CQ)],
                                  idxs.at[s, c], ssems[s]).wait()
            pltpu.make_async_copy(wgt_hbm[c].at[pl.ds(0, _CQ)],
                                  wgts.at[s, c], ssems[s]).wait()

    def fire_gathers(s):
        for c in range(4):
            for q in range(_CQ):
                pltpu.async_copy(v2.at[idxs.at[s, c, q]], rows.at[s, c, q],
                                 gsems[s])

    def wait_gathers(s):
        for c in range(4):
            for q in range(_CQ):
                pltpu.make_async_copy(v2.at[idxs.at[s, c, q]],
                                      rows.at[s, c, q], gsems[s]).wait()

    def compute(g, s):
        qb = base0 + g * _CQ
        for q in range(_CQ):
            def hbody(hh, carry2):
                acc0 = jnp.zeros((16,), jnp.float32)
                acc1 = jnp.zeros((16,), jnp.float32)
                hb = hh * 16
                for c in range(4):
                    wv = wgts[s, c, q, pl.ds(hb, 16)]
                    for k in range(16):
                        r = rows[s, c, q, hb + k, pl.ds(0, 32)]
                        r0, r1 = plsc.unpack(r, format=plsc.PackFormat.INTERLEAVED)
                        wb = wv[k]
                        acc0 = acc0 + wb * r0
                        acc1 = acc1 + wb * r1
                outv[pl.ds(q * _C + hh * 32, 16)] = acc0
                outv[pl.ds(q * _C + hh * 32 + 16, 16)] = acc1
                return carry2
            lax.fori_loop(0, _NH, hbody, 0)
        pltpu.sync_copy(outv, out.at[pl.ds(qb * _C, _CQ * _C)])

    fire_stage(0, 0)
    fire_stage(1, 1)
    wait_stage(0)
    fire_gathers(0)

    def pair(i, carry):
        for s in (0, 1):
            g = i * 2 + s
            o = 1 - s
            wait_gathers(s)

            @pl.when(g + 1 < _NCHUNK)
            def _():
                wait_stage(o)
                fire_gathers(o)

            compute(g, s)

            @pl.when(g + 2 < _NCHUNK)
            def _():
                fire_stage(g + 2, s)
        return carry

    lax.fori_loop(0, _NCHUNK // 2, pair, 0)


def _sc_msda(v2, i0, i1, i2, i3, w0, w1, w2, w3):
    mesh = plsc.VectorSubcoreMesh(core_axis_name="c", subcore_axis_name="s")
    return pl.kernel(
        _sc_body,
        out_type=jax.ShapeDtypeStruct((_NPAD * _C,), jnp.float32),
        mesh=mesh,
        compiler_params=pltpu.CompilerParams(needs_layout_passes=False,
                                             use_tc_tiling_on_sc=False),
        scratch_types=[
            pltpu.VMEM((2, 4, _CQ, 128), jnp.int32),
            pltpu.VMEM((2, 4, _CQ, 128), jnp.float32),
            pltpu.VMEM((2, 4, _CQ, 128, _DH), jnp.bfloat16),
            pltpu.VMEM((_CQ * _C,), jnp.float32),
            pltpu.SemaphoreType.DMA,
            pltpu.SemaphoreType.DMA,
            pltpu.SemaphoreType.DMA,
            pltpu.SemaphoreType.DMA,
        ],
    )(v2, i0, i1, i2, i3, w0, w1, w2, w3)


def _tail_body(ms_ref, q_ref, wo_ref, bo_ref, g1_ref, be1_ref,
               wl1_ref, bl1_ref, wl2_ref, bl2_ref, g2_ref, be2_ref, o_ref):
    f32 = jnp.float32
    src2 = jnp.dot(ms_ref[...], wo_ref[...], preferred_element_type=f32) + bo_ref[...]
    x = q_ref[...] + src2
    m = jnp.mean(x, axis=-1, keepdims=True)
    d = x - m
    v = jnp.mean(d * d, axis=-1, keepdims=True)
    q1 = d / jnp.sqrt(v + 1e-5) * g1_ref[...] + be1_ref[...]
    h1 = jnp.maximum(jnp.dot(q1, wl1_ref[...], preferred_element_type=f32)
                     + bl1_ref[...], 0.0)
    y = q1 + jnp.dot(h1, wl2_ref[...], preferred_element_type=f32) + bl2_ref[...]
    m2 = jnp.mean(y, axis=-1, keepdims=True)
    d2 = y - m2
    v2 = jnp.mean(d2 * d2, axis=-1, keepdims=True)
    o_ref[...] = d2 / jnp.sqrt(v2 + 1e-5) * g2_ref[...] + be2_ref[...]


def _full(shape):
    return pl.BlockSpec(shape, lambda i: tuple(0 for _ in shape))


def _prep_call(q, qp, v, rp8, W_val, b_val, Wx, bx, Wy, by, W_attw, b_attw):
    row = lambda i: (i, 0)
    outs = [jax.ShapeDtypeStruct((_N, _C), jnp.bfloat16)]
    outs += [jax.ShapeDtypeStruct((_NPAD, 128), jnp.int32)] * 4
    outs += [jax.ShapeDtypeStruct((_NPAD, 128), jnp.float32)] * 4
    out_specs = [pl.BlockSpec((_RB, _C), row)] + [pl.BlockSpec((_RB, 128), row)] * 8
    in_specs = [pl.BlockSpec((_RB, _C), row)] * 3 + [pl.BlockSpec((_RB, 8), row)]
    in_specs += [_full(a.shape) for a in
                 (W_val, b_val, Wx, bx, Wy, by, W_attw, b_attw,
                  _G, _PX, _PY, _WF, _HF, _WI, _ST, _HD)]
    return pl.pallas_call(
        _prep_body, grid=(_NBLK,), in_specs=in_specs,
        out_specs=out_specs, out_shape=outs,
    )(q, qp, v, rp8, W_val, b_val, Wx, bx, Wy, by, W_attw, b_attw,
      _G, _PX, _PY, _WF, _HF, _WI, _ST, _HD)


def _tail_call(ms, q, W_out, b_out, g1, be1, W_l1, b_l1, W_l2, b_l2, g2, be2):
    row = lambda i: (i, 0)
    full_args = (W_out, b_out, g1, be1, W_l1, b_l1, W_l2, b_l2, g2, be2)
    in_specs = [pl.BlockSpec((_RB, _C), row)] * 2 + [_full(a.shape) for a in full_args]
    return pl.pallas_call(
        _tail_body, grid=(_NBLK,), in_specs=in_specs,
        out_specs=pl.BlockSpec((_RB, _C), row),
        out_shape=jax.ShapeDtypeStruct((_N, _C), jnp.float32),
    )(ms, q, *full_args)


def kernel(query, query_pos, value, reference_points, spatial_shapes,
           level_start_index, score_tgt, foreground_pre_layer,
           W_val, b_val, W_off, b_off, W_attw, b_attw, W_out, b_out,
           g1, be1, W_l1, b_l1, W_l2, b_l2, g2, be2):
    q = query.reshape(_N, _C)
    qp = query_pos.reshape(_N, _C)
    v = value.reshape(_N, _C)
    rp8 = reference_points.reshape(_N, _NL * 2)
    Wx = W_off[:, 0::2]
    bx = b_off[0::2].reshape(1, 128)
    Wy = W_off[:, 1::2]
    by = b_off[1::2].reshape(1, 128)
    vp, i0, i1, i2, i3, w0, w1, w2, w3 = _prep_call(
        q, qp, v, rp8, W_val, b_val.reshape(1, _C), Wx, bx, Wy, by,
        W_attw, b_attw.reshape(1, 128))
    v2 = vp.reshape(_N * _NH, _DH)
    ms = _sc_msda(v2, i0, i1, i2, i3, w0, w1, w2, w3).reshape(_NPAD, _C)
    out = _tail_call(ms, q, W_out[_WOPERM], b_out.reshape(1, _C),
                     g1.reshape(1, _C), be1.reshape(1, _C),
                     W_l1, b_l1.reshape(1, _DFFN), W_l2, b_l2.reshape(1, _C),
                     g2.reshape(1, _C), be2.reshape(1, _C))
    return out.reshape(_B, _N, _C)
